# Initial kernel scaffold; baseline (speedup 1.0000x reference)
import jax
import jax.numpy as jnp
from jax.experimental import pallas as pl


def kernel(xyz1, xyz2, points1, points2, W1, b1, g1, be1, W2, b2, g2, be2):
    raise NotImplementedError("write your pallas kernel here")


# speedup vs baseline: 20.0999x; 20.0999x over previous
import jax
import jax.numpy as jnp
from jax import lax
from jax.experimental import pallas as pl

B, N, S = 8, 4096, 1024
D1, D2 = 64, 128
C_IN = D1 + D2
M0, M1 = 128, 128
P = B * N

TN = 256
TP = 2048
NB = N // TN
PB = P // TP

_BIG = 1e30
_HIGH = lax.Precision.HIGHEST


def _k1(xyz1_ref, xyz2t_ref, p1_ref, p2_ref, w1at_ref, w1bt_ref, b1_ref,
        h1_ref, sum_ref, sumsq_ref):
    a = xyz1_ref[0]
    bt = xyz2t_ref[0]
    dist = jnp.zeros((TN, S), jnp.float32)
    for d in range(3):
        diff = a[:, d:d + 1] - bt[d:d + 1, :]
        dist = dist + diff * diff

    iota = lax.broadcasted_iota(jnp.int32, (TN, S), 1)
    dcur = dist
    vals = []
    idxs = []
    for _ in range(3):
        mk = jnp.min(dcur, axis=1, keepdims=True)
        is_min = dcur == mk
        ik = jnp.min(jnp.where(is_min, iota, S), axis=1, keepdims=True)
        dcur = jnp.where(iota == ik, _BIG, dcur)
        vals.append(mk)
        idxs.append(ik)

    w0 = 1.0 / (vals[0] + 1e-8)
    w1 = 1.0 / (vals[1] + 1e-8)
    w2 = 1.0 / (vals[2] + 1e-8)
    wsum = w0 + w1 + w2
    wm = jnp.where(iota == idxs[0], w0 / wsum, 0.0)
    wm = wm + jnp.where(iota == idxs[1], w1 / wsum, 0.0)
    wm = wm + jnp.where(iota == idxs[2], w2 / wsum, 0.0)

    interp = jnp.dot(wm, p2_ref[0], preferred_element_type=jnp.float32,
                     precision=_HIGH)
    h1 = (jnp.dot(p1_ref[0], w1at_ref[...], preferred_element_type=jnp.float32,
                  precision=_HIGH)
          + jnp.dot(interp, w1bt_ref[...], preferred_element_type=jnp.float32,
                    precision=_HIGH)
          + b1_ref[...])
    h1_ref[0] = h1

    first = (pl.program_id(0) == 0) & (pl.program_id(1) == 0)

    @pl.when(first)
    def _():
        sum_ref[...] = jnp.zeros_like(sum_ref)
        sumsq_ref[...] = jnp.zeros_like(sumsq_ref)

    sum_ref[...] += jnp.sum(h1, axis=0, keepdims=True)
    sumsq_ref[...] += jnp.sum(h1 * h1, axis=0, keepdims=True)


def _bn(h, s, ss, g, be):
    m = s * (1.0 / P)
    v = ss * (1.0 / P) - m * m
    rstd = lax.rsqrt(v + 1e-5)
    return (h - m) * (rstd * g) + be


def _k2(h1_ref, s1_ref, ss1_ref, g1_ref, be1_ref, w2t_ref, b2_ref,
        sum_ref, sumsq_ref):
    a1 = jnp.maximum(_bn(h1_ref[...], s1_ref[...], ss1_ref[...],
                         g1_ref[...], be1_ref[...]), 0.0)
    h2 = jnp.dot(a1, w2t_ref[...], preferred_element_type=jnp.float32,
                 precision=_HIGH) + b2_ref[...]

    @pl.when(pl.program_id(0) == 0)
    def _():
        sum_ref[...] = jnp.zeros_like(sum_ref)
        sumsq_ref[...] = jnp.zeros_like(sumsq_ref)

    sum_ref[...] += jnp.sum(h2, axis=0, keepdims=True)
    sumsq_ref[...] += jnp.sum(h2 * h2, axis=0, keepdims=True)


def _k3(h1_ref, s1_ref, ss1_ref, g1_ref, be1_ref, w2t_ref, b2_ref,
        s2_ref, ss2_ref, g2_ref, be2_ref, out_ref):
    a1 = jnp.maximum(_bn(h1_ref[...], s1_ref[...], ss1_ref[...],
                         g1_ref[...], be1_ref[...]), 0.0)
    h2 = jnp.dot(a1, w2t_ref[...], preferred_element_type=jnp.float32,
                 precision=_HIGH) + b2_ref[...]
    out_ref[...] = jnp.maximum(_bn(h2, s2_ref[...], ss2_ref[...],
                                   g2_ref[...], be2_ref[...]), 0.0)


def kernel(xyz1, xyz2, points1, points2, W1, b1, g1, be1, W2, b2, g2, be2):
    xyz2t = jnp.transpose(xyz2, (0, 2, 1))
    w1at = jnp.transpose(W1[:, :D1])
    w1bt = jnp.transpose(W1[:, D1:])
    w2t = jnp.transpose(W2)
    row = lambda v: v.reshape(1, -1)

    h1, s1, ss1 = pl.pallas_call(
        _k1,
        grid=(B, NB),
        in_specs=[
            pl.BlockSpec((1, TN, 3), lambda b, n: (b, n, 0)),
            pl.BlockSpec((1, 3, S), lambda b, n: (b, 0, 0)),
            pl.BlockSpec((1, TN, D1), lambda b, n: (b, n, 0)),
            pl.BlockSpec((1, S, D2), lambda b, n: (b, 0, 0)),
            pl.BlockSpec((D1, M0), lambda b, n: (0, 0)),
            pl.BlockSpec((D2, M0), lambda b, n: (0, 0)),
            pl.BlockSpec((1, M0), lambda b, n: (0, 0)),
        ],
        out_specs=[
            pl.BlockSpec((1, TN, M0), lambda b, n: (b, n, 0)),
            pl.BlockSpec((1, M0), lambda b, n: (0, 0)),
            pl.BlockSpec((1, M0), lambda b, n: (0, 0)),
        ],
        out_shape=[
            jax.ShapeDtypeStruct((B, N, M0), jnp.float32),
            jax.ShapeDtypeStruct((1, M0), jnp.float32),
            jax.ShapeDtypeStruct((1, M0), jnp.float32),
        ],
    )(xyz1, xyz2t, points1, points2, w1at, w1bt, row(b1))

    h1f = h1.reshape(P, M0)

    s2, ss2 = pl.pallas_call(
        _k2,
        grid=(PB,),
        in_specs=[
            pl.BlockSpec((TP, M0), lambda i: (i, 0)),
            pl.BlockSpec((1, M0), lambda i: (0, 0)),
            pl.BlockSpec((1, M0), lambda i: (0, 0)),
            pl.BlockSpec((1, M0), lambda i: (0, 0)),
            pl.BlockSpec((1, M0), lambda i: (0, 0)),
            pl.BlockSpec((M0, M1), lambda i: (0, 0)),
            pl.BlockSpec((1, M1), lambda i: (0, 0)),
        ],
        out_specs=[
            pl.BlockSpec((1, M1), lambda i: (0, 0)),
            pl.BlockSpec((1, M1), lambda i: (0, 0)),
        ],
        out_shape=[
            jax.ShapeDtypeStruct((1, M1), jnp.float32),
            jax.ShapeDtypeStruct((1, M1), jnp.float32),
        ],
    )(h1f, s1, ss1, row(g1), row(be1), w2t, row(b2))

    out = pl.pallas_call(
        _k3,
        grid=(PB,),
        in_specs=[
            pl.BlockSpec((TP, M0), lambda i: (i, 0)),
            pl.BlockSpec((1, M0), lambda i: (0, 0)),
            pl.BlockSpec((1, M0), lambda i: (0, 0)),
            pl.BlockSpec((1, M0), lambda i: (0, 0)),
            pl.BlockSpec((1, M0), lambda i: (0, 0)),
            pl.BlockSpec((M0, M1), lambda i: (0, 0)),
            pl.BlockSpec((1, M1), lambda i: (0, 0)),
            pl.BlockSpec((1, M1), lambda i: (0, 0)),
            pl.BlockSpec((1, M1), lambda i: (0, 0)),
            pl.BlockSpec((1, M1), lambda i: (0, 0)),
            pl.BlockSpec((1, M1), lambda i: (0, 0)),
        ],
        out_specs=pl.BlockSpec((TP, M1), lambda i: (i, 0)),
        out_shape=jax.ShapeDtypeStruct((P, M1), jnp.float32),
    )(h1f, s1, ss1, row(g1), row(be1), w2t, row(b2),
      s2, ss2, row(g2), row(be2))

    return out.reshape(B, N, M1)


# speedup vs baseline: 36.0875x; 1.7954x over previous
import jax
import jax.numpy as jnp
from jax import lax
from jax.experimental import pallas as pl
from jax.experimental.pallas import tpu as pltpu

B, N, S = 8, 4096, 1024
D1, D2 = 64, 128
C_IN = D1 + D2
M0, M1 = 128, 128
P = B * N

TN = 512
TP = 2048
NB = N // TN
PB = P // TP
SA = B * NB
ST = SA + 2 * PB

_BIG = 1e30


def _bn(h, s, ss, g, be):
    m = s * (1.0 / P)
    v = ss * (1.0 / P) - m * m
    rstd = lax.rsqrt(v + 1e-5)
    return (h - m) * (rstd * g) + be


def _fused(xyz1_ref, xyz2t_ref, p1_ref, p2_ref, w1at_ref, w1bt_ref, b1_ref,
           g1_ref, be1_ref, w2t_ref, b2_ref, g2_ref, be2_ref,
           out_ref, h1s, s1_ref, ss1_ref, s2_ref, ss2_ref):
    t = pl.program_id(0)

    @pl.when(t == 0)
    def _():
        s1_ref[...] = jnp.zeros_like(s1_ref)
        ss1_ref[...] = jnp.zeros_like(ss1_ref)
        s2_ref[...] = jnp.zeros_like(s2_ref)
        ss2_ref[...] = jnp.zeros_like(ss2_ref)

    @pl.when(t < SA)
    def _phase_a():
        a = xyz1_ref[0]
        bt = xyz2t_ref[0]
        xx = jnp.sum(a * a, axis=1, keepdims=True)
        yy = jnp.sum(bt * bt, axis=0, keepdims=True)
        cross = jnp.dot(a, bt, preferred_element_type=jnp.float32,
                        precision=lax.Precision.HIGHEST)
        dist = jnp.maximum((xx + yy) - (cross + cross), 0.0)

        dcur = dist
        wm = None
        wsum = None
        for k in range(3):
            mk = jnp.min(dcur, axis=1, keepdims=True)
            eq = dcur == mk
            wk = 1.0 / (mk + 1e-8)
            wm = jnp.where(eq, wk, 0.0 if k == 0 else wm)
            dcur = jnp.where(eq, _BIG, dcur)
            wsum = wk if k == 0 else wsum + wk

        interp = jnp.dot(wm, p2_ref[0],
                         preferred_element_type=jnp.float32) * (1.0 / wsum)
        h1 = (jnp.dot(p1_ref[0], w1at_ref[...],
                      preferred_element_type=jnp.float32)
              + jnp.dot(interp, w1bt_ref[...],
                        preferred_element_type=jnp.float32)
              + b1_ref[...])
        h1s[pl.ds(pl.multiple_of(t * TN, TN), TN), :] = h1
        s1_ref[...] += jnp.sum(h1, axis=0, keepdims=True)
        ss1_ref[...] += jnp.sum(h1 * h1, axis=0, keepdims=True)

    @pl.when((t >= SA) & (t < SA + PB))
    def _phase_b():
        off = pl.multiple_of((t - SA) * TP, TP)
        h1 = h1s[pl.ds(off, TP), :]
        a1 = jnp.maximum(_bn(h1, s1_ref[...], ss1_ref[...],
                             g1_ref[...], be1_ref[...]), 0.0)
        h2 = jnp.dot(a1, w2t_ref[...],
                     preferred_element_type=jnp.float32) + b2_ref[...]
        s2_ref[...] += jnp.sum(h2, axis=0, keepdims=True)
        ss2_ref[...] += jnp.sum(h2 * h2, axis=0, keepdims=True)

    @pl.when(t >= SA + PB)
    def _phase_c():
        off = pl.multiple_of((t - SA - PB) * TP, TP)
        h1 = h1s[pl.ds(off, TP), :]
        a1 = jnp.maximum(_bn(h1, s1_ref[...], ss1_ref[...],
                             g1_ref[...], be1_ref[...]), 0.0)
        h2 = jnp.dot(a1, w2t_ref[...],
                     preferred_element_type=jnp.float32) + b2_ref[...]
        out_ref[...] = jnp.maximum(_bn(h2, s2_ref[...], ss2_ref[...],
                                       g2_ref[...], be2_ref[...]), 0.0)


def kernel(xyz1, xyz2, points1, points2, W1, b1, g1, be1, W2, b2, g2, be2):
    xyz2t = jnp.transpose(xyz2, (0, 2, 1))
    w1at = jnp.transpose(W1[:, :D1])
    w1bt = jnp.transpose(W1[:, D1:])
    w2t = jnp.transpose(W2)
    row = lambda v: v.reshape(1, -1)

    def amap(t):
        ta = jnp.minimum(t, SA - 1)
        return ta // NB, ta % NB

    out = pl.pallas_call(
        _fused,
        grid=(ST,),
        in_specs=[
            pl.BlockSpec((1, TN, 3), lambda t: (amap(t)[0], amap(t)[1], 0)),
            pl.BlockSpec((1, 3, S), lambda t: (amap(t)[0], 0, 0)),
            pl.BlockSpec((1, TN, D1), lambda t: (amap(t)[0], amap(t)[1], 0)),
            pl.BlockSpec((1, S, D2), lambda t: (amap(t)[0], 0, 0)),
            pl.BlockSpec((D1, M0), lambda t: (0, 0)),
            pl.BlockSpec((D2, M0), lambda t: (0, 0)),
            pl.BlockSpec((1, M0), lambda t: (0, 0)),
            pl.BlockSpec((1, M0), lambda t: (0, 0)),
            pl.BlockSpec((1, M0), lambda t: (0, 0)),
            pl.BlockSpec((M0, M1), lambda t: (0, 0)),
            pl.BlockSpec((1, M1), lambda t: (0, 0)),
            pl.BlockSpec((1, M1), lambda t: (0, 0)),
            pl.BlockSpec((1, M1), lambda t: (0, 0)),
        ],
        out_specs=pl.BlockSpec(
            (TP, M1), lambda t: (jnp.maximum(t - SA - PB, 0), 0)),
        out_shape=jax.ShapeDtypeStruct((P, M1), jnp.float32),
        scratch_shapes=[
            pltpu.VMEM((P, M0), jnp.float32),
            pltpu.VMEM((1, M0), jnp.float32),
            pltpu.VMEM((1, M0), jnp.float32),
            pltpu.VMEM((1, M1), jnp.float32),
            pltpu.VMEM((1, M1), jnp.float32),
        ],
    )(xyz1, xyz2t, points1, points2, w1at, w1bt, row(b1),
      row(g1), row(be1), w2t, row(b2), row(g2), row(be2))

    return out.reshape(B, N, M1)


# speedup vs baseline: 49.9012x; 1.3828x over previous
import jax
import jax.numpy as jnp
from jax import lax
from jax.experimental import pallas as pl
from jax.experimental.pallas import tpu as pltpu

B, N, S = 8, 4096, 1024
D1, D2 = 64, 128
C_IN = D1 + D2
M0, M1 = 128, 128
P = B * N

TN = 512
TP = 2048
NB = N // TN
PB = P // TP
SA = B * NB
ST = SA + 2 * PB

_BIG = 1e30


def _bn(h, s, ss, g, be):
    m = s * (1.0 / P)
    v = ss * (1.0 / P) - m * m
    rstd = lax.rsqrt(v + 1e-5)
    return (h - m) * (rstd * g) + be


def _fused(xyz1_ref, xyz2t_ref, p1_ref, p2_ref, w1at_ref, w1bt_ref, b1_ref,
           g1_ref, be1_ref, w2t_ref, b2_ref, g2_ref, be2_ref,
           out_ref, h1s, s1_ref, ss1_ref, s2_ref, ss2_ref):
    t = pl.program_id(0)

    @pl.when(t == 0)
    def _():
        s1_ref[...] = jnp.zeros_like(s1_ref)
        ss1_ref[...] = jnp.zeros_like(ss1_ref)
        s2_ref[...] = jnp.zeros_like(s2_ref)
        ss2_ref[...] = jnp.zeros_like(ss2_ref)

    @pl.when(t < SA)
    def _phase_a():
        a = xyz1_ref[0]
        bt = xyz2t_ref[0]
        dist = jnp.zeros((TN, S), jnp.float32)
        for d in range(3):
            diff = a[:, d:d + 1] - bt[d:d + 1, :]
            dist = dist + diff * diff

        dcur = dist
        wm = None
        wsum = None
        for k in range(3):
            mk = jnp.min(dcur, axis=1, keepdims=True)
            eq = dcur == mk
            wk = 1.0 / (mk + 1e-8)
            wm = jnp.where(eq, wk, 0.0 if k == 0 else wm)
            dcur = jnp.where(eq, _BIG, dcur)
            wsum = wk if k == 0 else wsum + wk

        interp = jnp.dot(wm, p2_ref[0],
                         preferred_element_type=jnp.float32) * (1.0 / wsum)
        h1 = (jnp.dot(p1_ref[0], w1at_ref[...],
                      preferred_element_type=jnp.float32)
              + jnp.dot(interp, w1bt_ref[...],
                        preferred_element_type=jnp.float32)
              + b1_ref[...])
        h1s[pl.ds(pl.multiple_of(t * TN, TN), TN), :] = h1
        s1_ref[...] += jnp.sum(h1, axis=0, keepdims=True)
        ss1_ref[...] += jnp.sum(h1 * h1, axis=0, keepdims=True)

    @pl.when((t >= SA) & (t < SA + PB))
    def _phase_b():
        off = pl.multiple_of((t - SA) * TP, TP)
        h1 = h1s[pl.ds(off, TP), :]
        a1 = jnp.maximum(_bn(h1, s1_ref[...], ss1_ref[...],
                             g1_ref[...], be1_ref[...]), 0.0)
        h2 = jnp.dot(a1, w2t_ref[...],
                     preferred_element_type=jnp.float32) + b2_ref[...]
        s2_ref[...] += jnp.sum(h2, axis=0, keepdims=True)
        ss2_ref[...] += jnp.sum(h2 * h2, axis=0, keepdims=True)

    @pl.when(t >= SA + PB)
    def _phase_c():
        off = pl.multiple_of((t - SA - PB) * TP, TP)
        h1 = h1s[pl.ds(off, TP), :]
        a1 = jnp.maximum(_bn(h1, s1_ref[...], ss1_ref[...],
                             g1_ref[...], be1_ref[...]), 0.0)
        h2 = jnp.dot(a1, w2t_ref[...],
                     preferred_element_type=jnp.float32) + b2_ref[...]
        out_ref[...] = jnp.maximum(_bn(h2, s2_ref[...], ss2_ref[...],
                                       g2_ref[...], be2_ref[...]), 0.0)


def kernel(xyz1, xyz2, points1, points2, W1, b1, g1, be1, W2, b2, g2, be2):
    xyz2t = jnp.transpose(xyz2, (0, 2, 1))
    w1at = jnp.transpose(W1[:, :D1])
    w1bt = jnp.transpose(W1[:, D1:])
    w2t = jnp.transpose(W2)
    row = lambda v: v.reshape(1, -1)

    def amap(t):
        ta = jnp.minimum(t, SA - 1)
        return ta // NB, ta % NB

    out = pl.pallas_call(
        _fused,
        grid=(ST,),
        in_specs=[
            pl.BlockSpec((1, TN, 3), lambda t: (amap(t)[0], amap(t)[1], 0)),
            pl.BlockSpec((1, 3, S), lambda t: (amap(t)[0], 0, 0)),
            pl.BlockSpec((1, TN, D1), lambda t: (amap(t)[0], amap(t)[1], 0)),
            pl.BlockSpec((1, S, D2), lambda t: (amap(t)[0], 0, 0)),
            pl.BlockSpec((D1, M0), lambda t: (0, 0)),
            pl.BlockSpec((D2, M0), lambda t: (0, 0)),
            pl.BlockSpec((1, M0), lambda t: (0, 0)),
            pl.BlockSpec((1, M0), lambda t: (0, 0)),
            pl.BlockSpec((1, M0), lambda t: (0, 0)),
            pl.BlockSpec((M0, M1), lambda t: (0, 0)),
            pl.BlockSpec((1, M1), lambda t: (0, 0)),
            pl.BlockSpec((1, M1), lambda t: (0, 0)),
            pl.BlockSpec((1, M1), lambda t: (0, 0)),
        ],
        out_specs=pl.BlockSpec(
            (TP, M1), lambda t: (jnp.maximum(t - SA - PB, 0), 0)),
        out_shape=jax.ShapeDtypeStruct((P, M1), jnp.float32),
        scratch_shapes=[
            pltpu.VMEM((P, M0), jnp.float32),
            pltpu.VMEM((1, M0), jnp.float32),
            pltpu.VMEM((1, M0), jnp.float32),
            pltpu.VMEM((1, M1), jnp.float32),
            pltpu.VMEM((1, M1), jnp.float32),
        ],
    )(xyz1, xyz2t, points1, points2, w1at, w1bt, row(b1),
      row(g1), row(be1), w2t, row(b2), row(g2), row(be2))

    return out.reshape(B, N, M1)


# speedup vs baseline: 51.9602x; 1.0413x over previous
import jax
import jax.numpy as jnp
from jax import lax
from jax.experimental import pallas as pl
from jax.experimental.pallas import tpu as pltpu

B, N, S = 8, 4096, 1024
D1, D2 = 64, 128
C_IN = D1 + D2
M0, M1 = 128, 128
P = B * N

TN = 1024
TP = 2048
NB = N // TN
PB = P // TP
SA = B * NB
ST = SA + 2 * PB

_BIG = 1e30


def _bn(h, s, ss, g, be):
    m = s * (1.0 / P)
    v = ss * (1.0 / P) - m * m
    rstd = lax.rsqrt(v + 1e-5)
    return (h - m) * (rstd * g) + be


def _fused(xyz1_ref, xyz2t_ref, p1_ref, p2_ref, w1at_ref, w1bt_ref, b1_ref,
           g1_ref, be1_ref, w2t_ref, b2_ref, g2_ref, be2_ref,
           out_ref, h1s, s1_ref, ss1_ref, s2_ref, ss2_ref):
    t = pl.program_id(0)

    @pl.when(t == 0)
    def _():
        s1_ref[...] = jnp.zeros_like(s1_ref)
        ss1_ref[...] = jnp.zeros_like(ss1_ref)
        s2_ref[...] = jnp.zeros_like(s2_ref)
        ss2_ref[...] = jnp.zeros_like(ss2_ref)

    @pl.when(t < SA)
    def _phase_a():
        a = xyz1_ref[0]
        bt = xyz2t_ref[0]
        dist = jnp.zeros((TN, S), jnp.float32)
        for d in range(3):
            diff = a[:, d:d + 1] - bt[d:d + 1, :]
            dist = dist + diff * diff

        dcur = dist
        wm = None
        wsum = None
        for k in range(3):
            mk = jnp.min(dcur, axis=1, keepdims=True)
            eq = dcur == mk
            wk = 1.0 / (mk + 1e-8)
            wm = jnp.where(eq, wk, 0.0 if k == 0 else wm)
            dcur = jnp.where(eq, _BIG, dcur)
            wsum = wk if k == 0 else wsum + wk

        interp = jnp.dot(wm, p2_ref[0],
                         preferred_element_type=jnp.float32) * (1.0 / wsum)
        h1 = (jnp.dot(p1_ref[0], w1at_ref[...],
                      preferred_element_type=jnp.float32)
              + jnp.dot(interp, w1bt_ref[...],
                        preferred_element_type=jnp.float32)
              + b1_ref[...])
        h1s[pl.ds(pl.multiple_of(t * TN, TN), TN), :] = h1
        s1_ref[...] += jnp.sum(h1, axis=0, keepdims=True)
        ss1_ref[...] += jnp.sum(h1 * h1, axis=0, keepdims=True)

    @pl.when((t >= SA) & (t < SA + PB))
    def _phase_b():
        off = pl.multiple_of((t - SA) * TP, TP)
        h1 = h1s[pl.ds(off, TP), :]
        a1 = jnp.maximum(_bn(h1, s1_ref[...], ss1_ref[...],
                             g1_ref[...], be1_ref[...]), 0.0)
        h2 = jnp.dot(a1, w2t_ref[...],
                     preferred_element_type=jnp.float32) + b2_ref[...]
        s2_ref[...] += jnp.sum(h2, axis=0, keepdims=True)
        ss2_ref[...] += jnp.sum(h2 * h2, axis=0, keepdims=True)

    @pl.when(t >= SA + PB)
    def _phase_c():
        off = pl.multiple_of((t - SA - PB) * TP, TP)
        h1 = h1s[pl.ds(off, TP), :]
        a1 = jnp.maximum(_bn(h1, s1_ref[...], ss1_ref[...],
                             g1_ref[...], be1_ref[...]), 0.0)
        h2 = jnp.dot(a1, w2t_ref[...],
                     preferred_element_type=jnp.float32) + b2_ref[...]
        out_ref[...] = jnp.maximum(_bn(h2, s2_ref[...], ss2_ref[...],
                                       g2_ref[...], be2_ref[...]), 0.0)


def kernel(xyz1, xyz2, points1, points2, W1, b1, g1, be1, W2, b2, g2, be2):
    xyz2t = jnp.transpose(xyz2, (0, 2, 1))
    w1at = jnp.transpose(W1[:, :D1])
    w1bt = jnp.transpose(W1[:, D1:])
    w2t = jnp.transpose(W2)
    row = lambda v: v.reshape(1, -1)

    def amap(t):
        ta = jnp.minimum(t, SA - 1)
        return ta // NB, ta % NB

    out = pl.pallas_call(
        _fused,
        grid=(ST,),
        in_specs=[
            pl.BlockSpec((1, TN, 3), lambda t: (amap(t)[0], amap(t)[1], 0)),
            pl.BlockSpec((1, 3, S), lambda t: (amap(t)[0], 0, 0)),
            pl.BlockSpec((1, TN, D1), lambda t: (amap(t)[0], amap(t)[1], 0)),
            pl.BlockSpec((1, S, D2), lambda t: (amap(t)[0], 0, 0)),
            pl.BlockSpec((D1, M0), lambda t: (0, 0)),
            pl.BlockSpec((D2, M0), lambda t: (0, 0)),
            pl.BlockSpec((1, M0), lambda t: (0, 0)),
            pl.BlockSpec((1, M0), lambda t: (0, 0)),
            pl.BlockSpec((1, M0), lambda t: (0, 0)),
            pl.BlockSpec((M0, M1), lambda t: (0, 0)),
            pl.BlockSpec((1, M1), lambda t: (0, 0)),
            pl.BlockSpec((1, M1), lambda t: (0, 0)),
            pl.BlockSpec((1, M1), lambda t: (0, 0)),
        ],
        out_specs=pl.BlockSpec(
            (TP, M1), lambda t: (jnp.maximum(t - SA - PB, 0), 0)),
        out_shape=jax.ShapeDtypeStruct((P, M1), jnp.float32),
        scratch_shapes=[
            pltpu.VMEM((P, M0), jnp.float32),
            pltpu.VMEM((1, M0), jnp.float32),
            pltpu.VMEM((1, M0), jnp.float32),
            pltpu.VMEM((1, M1), jnp.float32),
            pltpu.VMEM((1, M1), jnp.float32),
        ],
    )(xyz1, xyz2t, points1, points2, w1at, w1bt, row(b1),
      row(g1), row(be1), w2t, row(b2), row(g2), row(be2))

    return out.reshape(B, N, M1)


# speedup vs baseline: 53.4018x; 1.0277x over previous
import jax
import jax.numpy as jnp
from jax import lax
from jax.experimental import pallas as pl
from jax.experimental.pallas import tpu as pltpu

B, N, S = 8, 4096, 1024
D1, D2 = 64, 128
C_IN = D1 + D2
M0, M1 = 128, 128
P = B * N

TN = 1024
TP = 4096
NB = N // TN
PB = P // TP
SA = B * NB
ST = SA + 2 * PB

_BIG = 1e30


def _bn(h, s, ss, g, be):
    m = s * (1.0 / P)
    v = ss * (1.0 / P) - m * m
    rstd = lax.rsqrt(v + 1e-5)
    return (h - m) * (rstd * g) + be


def _fused(xyz1_ref, xyz2t_ref, p1_ref, p2_ref, w1at_ref, w1bt_ref, b1_ref,
           g1_ref, be1_ref, w2t_ref, b2_ref, g2_ref, be2_ref,
           out_ref, h1s, s1_ref, ss1_ref, s2_ref, ss2_ref):
    t = pl.program_id(0)

    @pl.when(t == 0)
    def _():
        s1_ref[...] = jnp.zeros_like(s1_ref)
        ss1_ref[...] = jnp.zeros_like(ss1_ref)
        s2_ref[...] = jnp.zeros_like(s2_ref)
        ss2_ref[...] = jnp.zeros_like(ss2_ref)

    @pl.when(t < SA)
    def _phase_a():
        a = xyz1_ref[0]
        bt = xyz2t_ref[0]
        dist = jnp.zeros((TN, S), jnp.float32)
        for d in range(3):
            diff = a[:, d:d + 1] - bt[d:d + 1, :]
            dist = dist + diff * diff

        dcur = dist
        wm = None
        wsum = None
        for k in range(3):
            mk = jnp.min(dcur, axis=1, keepdims=True)
            eq = dcur == mk
            wk = 1.0 / (mk + 1e-8)
            wm = jnp.where(eq, wk, 0.0 if k == 0 else wm)
            if k < 2:
                dcur = jnp.where(eq, _BIG, dcur)
            wsum = wk if k == 0 else wsum + wk

        interp = jnp.dot(wm, p2_ref[0],
                         preferred_element_type=jnp.float32) * (1.0 / wsum)
        h1 = (jnp.dot(p1_ref[0], w1at_ref[...],
                      preferred_element_type=jnp.float32)
              + jnp.dot(interp, w1bt_ref[...],
                        preferred_element_type=jnp.float32)
              + b1_ref[...])
        h1s[pl.ds(pl.multiple_of(t * TN, TN), TN), :] = h1
        s1_ref[...] += jnp.sum(h1, axis=0, keepdims=True)
        ss1_ref[...] += jnp.sum(h1 * h1, axis=0, keepdims=True)

    @pl.when((t >= SA) & (t < SA + PB))
    def _phase_b():
        off = pl.multiple_of((t - SA) * TP, TP)
        h1 = h1s[pl.ds(off, TP), :]
        a1 = jnp.maximum(_bn(h1, s1_ref[...], ss1_ref[...],
                             g1_ref[...], be1_ref[...]), 0.0)
        h2 = jnp.dot(a1, w2t_ref[...],
                     preferred_element_type=jnp.float32) + b2_ref[...]
        s2_ref[...] += jnp.sum(h2, axis=0, keepdims=True)
        ss2_ref[...] += jnp.sum(h2 * h2, axis=0, keepdims=True)

    @pl.when(t >= SA + PB)
    def _phase_c():
        off = pl.multiple_of((t - SA - PB) * TP, TP)
        h1 = h1s[pl.ds(off, TP), :]
        a1 = jnp.maximum(_bn(h1, s1_ref[...], ss1_ref[...],
                             g1_ref[...], be1_ref[...]), 0.0)
        h2 = jnp.dot(a1, w2t_ref[...],
                     preferred_element_type=jnp.float32) + b2_ref[...]
        out_ref[...] = jnp.maximum(_bn(h2, s2_ref[...], ss2_ref[...],
                                       g2_ref[...], be2_ref[...]), 0.0)


def kernel(xyz1, xyz2, points1, points2, W1, b1, g1, be1, W2, b2, g2, be2):
    xyz2t = jnp.transpose(xyz2, (0, 2, 1))
    w1at = jnp.transpose(W1[:, :D1])
    w1bt = jnp.transpose(W1[:, D1:])
    w2t = jnp.transpose(W2)
    row = lambda v: v.reshape(1, -1)

    def amap(t):
        ta = jnp.minimum(t, SA - 1)
        return ta // NB, ta % NB

    out = pl.pallas_call(
        _fused,
        grid=(ST,),
        in_specs=[
            pl.BlockSpec((1, TN, 3), lambda t: (amap(t)[0], amap(t)[1], 0)),
            pl.BlockSpec((1, 3, S), lambda t: (amap(t)[0], 0, 0)),
            pl.BlockSpec((1, TN, D1), lambda t: (amap(t)[0], amap(t)[1], 0)),
            pl.BlockSpec((1, S, D2), lambda t: (amap(t)[0], 0, 0)),
            pl.BlockSpec((D1, M0), lambda t: (0, 0)),
            pl.BlockSpec((D2, M0), lambda t: (0, 0)),
            pl.BlockSpec((1, M0), lambda t: (0, 0)),
            pl.BlockSpec((1, M0), lambda t: (0, 0)),
            pl.BlockSpec((1, M0), lambda t: (0, 0)),
            pl.BlockSpec((M0, M1), lambda t: (0, 0)),
            pl.BlockSpec((1, M1), lambda t: (0, 0)),
            pl.BlockSpec((1, M1), lambda t: (0, 0)),
            pl.BlockSpec((1, M1), lambda t: (0, 0)),
        ],
        out_specs=pl.BlockSpec(
            (TP, M1), lambda t: (jnp.maximum(t - SA - PB, 0), 0)),
        out_shape=jax.ShapeDtypeStruct((P, M1), jnp.float32),
        scratch_shapes=[
            pltpu.VMEM((P, M0), jnp.float32),
            pltpu.VMEM((1, M0), jnp.float32),
            pltpu.VMEM((1, M0), jnp.float32),
            pltpu.VMEM((1, M1), jnp.float32),
            pltpu.VMEM((1, M1), jnp.float32),
        ],
    )(xyz1, xyz2t, points1, points2, w1at, w1bt, row(b1),
      row(g1), row(be1), w2t, row(b2), row(g2), row(be2))

    return out.reshape(B, N, M1)
